# baseline (speedup 1.0000x reference)
import numpy as np
import jax
import jax.numpy as jnp
from jax.experimental import pallas as pl
from jax.experimental.pallas import tpu as pltpu

_BETA_1 = 1e-4
_BETA_T = 0.02
_T = 1000
_IMG = 32
_KPAD = 1024
_BB = 512


def _schedule_constants():
    betas = np.linspace(_BETA_1, _BETA_T, _T + 1, dtype=np.float64)
    s = np.sqrt(np.cumprod(1.0 - betas))
    P = np.cumprod(s)
    C = np.empty(_T + 1, dtype=np.float64)
    c = 0.0
    for k in range(_T + 1):
        c = c * s[k] + betas[k] * betas[k]
        C[k] = c
    Ppad = np.zeros((1, _KPAD), np.float32)
    Cpad = np.zeros((1, _KPAD), np.float32)
    Ppad[0, : _T + 1] = P
    Cpad[0, : _T + 1] = C
    return Ppad, Cpad


_P_ROW, _C_ROW = _schedule_constants()

_EYE_ROW = np.zeros((1, 3 * _IMG * _IMG), np.float32)
for _c in range(3):
    for _i in range(_IMG):
        _EYE_ROW[0, _c * _IMG * _IMG + _i * (_IMG + 1)] = 1.0


def _body(ts_ref, p_ref, c_ref, m_ref, x_ref, n_ref, o_ref):
    idx = ts_ref[0] - 1
    k = jax.lax.broadcasted_iota(jnp.int32, (_BB, _KPAD), 1)
    sel = idx == k
    pt = jnp.sum(jnp.where(sel, p_ref[...], 0.0), axis=1, keepdims=True)
    ct = jnp.sum(jnp.where(sel, c_ref[...], 0.0), axis=1, keepdims=True)
    o_ref[...] = x_ref[...] * pt + n_ref[...] * (m_ref[...] * ct)


def kernel(x_0, normal, timesteps):
    B = x_0.shape[0]
    D = x_0.shape[1] * x_0.shape[2] * x_0.shape[3]
    nb = B // _BB
    x = x_0.reshape(B, D)
    n = normal.reshape(B, D)
    ts3 = timesteps.reshape(nb, _BB, 1)
    p = jnp.asarray(_P_ROW)
    c = jnp.asarray(_C_ROW)
    m = jnp.asarray(_EYE_ROW)
    out = pl.pallas_call(
        _body,
        grid=(nb,),
        in_specs=[
            pl.BlockSpec((1, _BB, 1), lambda b: (b, 0, 0)),
            pl.BlockSpec((1, _KPAD), lambda b: (0, 0)),
            pl.BlockSpec((1, _KPAD), lambda b: (0, 0)),
            pl.BlockSpec((1, D), lambda b: (0, 0)),
            pl.BlockSpec((_BB, D), lambda b: (b, 0)),
            pl.BlockSpec((_BB, D), lambda b: (b, 0)),
        ],
        out_specs=pl.BlockSpec((_BB, D), lambda b: (b, 0)),
        out_shape=jax.ShapeDtypeStruct((B, D), jnp.float32),
        compiler_params=pltpu.CompilerParams(
            dimension_semantics=("parallel",),
        ),
    )(ts3, p, c, m, x, n)
    return out.reshape(x_0.shape)


# speedup vs baseline: 1.0040x; 1.0040x over previous
import numpy as np
import jax
import jax.numpy as jnp
from jax.experimental import pallas as pl
from jax.experimental.pallas import tpu as pltpu

_BETA_1 = 1e-4
_BETA_T = 0.02
_T = 1000
_IMG = 32
_KPAD = 1024
_BB = 512


def _schedule_constants():
    betas = np.linspace(_BETA_1, _BETA_T, _T + 1, dtype=np.float64)
    s = np.sqrt(np.cumprod(1.0 - betas))
    P = np.cumprod(s)
    C = np.empty(_T + 1, dtype=np.float64)
    c = 0.0
    for k in range(_T + 1):
        c = c * s[k] + betas[k] * betas[k]
        C[k] = c
    Ppad = np.zeros((1, _KPAD), np.float32)
    Cpad = np.zeros((1, _KPAD), np.float32)
    Ppad[0, : _T + 1] = P
    Cpad[0, : _T + 1] = C
    return Ppad, Cpad


_P_ROW, _C_ROW = _schedule_constants()

_EYE_ROW = np.zeros((1, 3 * _IMG * _IMG), np.float32)
for _c in range(3):
    for _i in range(_IMG):
        _EYE_ROW[0, _c * _IMG * _IMG + _i * (_IMG + 1)] = 1.0


def _body(ts_ref, p_ref, c_ref, m_ref, x_ref, n_ref, o_ref):
    idx = ts_ref[pl.program_id(0)] - 1
    k = jax.lax.broadcasted_iota(jnp.int32, (_BB, _KPAD), 1)
    sel = idx == k
    pt = jnp.sum(jnp.where(sel, p_ref[...], 0.0), axis=1, keepdims=True)
    ct = jnp.sum(jnp.where(sel, c_ref[...], 0.0), axis=1, keepdims=True)
    o_ref[...] = x_ref[...] * pt + n_ref[...] * (m_ref[...] * ct)


def kernel(x_0, normal, timesteps):
    B = x_0.shape[0]
    D = x_0.shape[1] * x_0.shape[2] * x_0.shape[3]
    nb = B // _BB
    x = x_0.reshape(B, D)
    n = normal.reshape(B, D)
    ts3 = timesteps.reshape(nb, _BB, 1)
    p = jnp.asarray(_P_ROW)
    c = jnp.asarray(_C_ROW)
    m = jnp.asarray(_EYE_ROW)
    out = pl.pallas_call(
        _body,
        grid=(nb,),
        in_specs=[
            pl.BlockSpec((B // _BB, _BB, 1), lambda b: (0, 0, 0)),
            pl.BlockSpec((1, _KPAD), lambda b: (0, 0)),
            pl.BlockSpec((1, _KPAD), lambda b: (0, 0)),
            pl.BlockSpec((1, D), lambda b: (0, 0)),
            pl.BlockSpec((_BB, D), lambda b: (b, 0)),
            pl.BlockSpec((_BB, D), lambda b: (b, 0)),
        ],
        out_specs=pl.BlockSpec((_BB, D), lambda b: (b, 0)),
        out_shape=jax.ShapeDtypeStruct((B, D), jnp.float32),
        compiler_params=pltpu.CompilerParams(
            dimension_semantics=("parallel",),
        ),
    )(ts3, p, c, m, x, n)
    return out.reshape(x_0.shape)
